# baseline (speedup 1.0000x reference)
import functools

import jax
import jax.numpy as jnp
from jax import lax
from jax.experimental import pallas as pl
from jax.experimental.pallas import tpu as pltpu
from jax.experimental.pallas import tpu_sc as plsc

A, E, H = 10000, 320000, 128
NC, NS = 2, 16
NW = NC * NS
CH = 128

SLICE_NCH = (20, 20, 20, 18)
SLICE_EXTRA = (0, 0, 0, 4)
SLICE_ES = tuple(NW * n * CH + x * CH
                 for n, x in zip(SLICE_NCH, SLICE_EXTRA))

RPS = 624
TAIL0 = NS * RPS
TAILN = A - TAIL0


@functools.cache
def _sc_kernels():
    mesh = plsc.VectorSubcoreMesh(core_axis_name="c", subcore_axis_name="s")

    def make_gather(nch, nextra):
        es = NW * nch * CH + nextra * CH

        @functools.partial(
            pl.kernel,
            out_type=(jax.ShapeDtypeStruct((es, H), jnp.float32),
                      jax.ShapeDtypeStruct((es, H), jnp.float32)),
            mesh=mesh,
            scratch_types=[
                pltpu.VMEM((2, CH), jnp.int32),
                pltpu.VMEM((2, CH), jnp.int32),
                pltpu.VMEM((CH, H), jnp.float32),
                pltpu.VMEM((CH, H), jnp.float32),
                pltpu.VMEM((CH, H), jnp.float32),
                pltpu.VMEM((CH, H), jnp.float32),
                pltpu.SemaphoreType.DMA,
                pltpu.SemaphoreType.DMA,
                pltpu.SemaphoreType.DMA,
                pltpu.SemaphoreType.DMA,
            ],
        )
        def gather(node_hbm, send_hbm, recv_hbm, sout_hbm, rout_hbm,
                   sidx_v, ridx_v, sr0, sr1, rr0, rr1,
                   sg0, sg1, sw0, sw1):
            c = lax.axis_index("c")
            s = lax.axis_index("s")
            wid = s * NC + c
            base = wid * nch * CH
            srow = (sr0, sr1)
            rrow = (rr0, rr1)
            sgs = (sg0, sg1)
            sws = (sw0, sw1)

            def load_idx(off, b):
                pltpu.sync_copy(send_hbm.at[pl.ds(off, CH)], sidx_v.at[b])
                pltpu.sync_copy(recv_hbm.at[pl.ds(off, CH)], ridx_v.at[b])

            def fire_gather(b):
                pltpu.async_copy(node_hbm.at[sidx_v.at[b]], srow[b], sgs[b])
                pltpu.async_copy(node_hbm.at[ridx_v.at[b]], rrow[b], sgs[b])

            def wait_gather(b):
                pltpu.make_async_copy(node_hbm.at[sidx_v.at[b]], srow[b],
                                      sgs[b]).wait()
                pltpu.make_async_copy(node_hbm.at[ridx_v.at[b]], rrow[b],
                                      sgs[b]).wait()

            load_idx(base, 0)
            fire_gather(0)
            load_idx(base + CH, 1)
            fire_gather(1)

            @pl.loop(0, (nch - 2) // 2)
            def _(j):
                for b in (0, 1):
                    off = base + (2 * j + b) * CH
                    wait_gather(b)
                    w1 = pltpu.async_copy(srow[b], sout_hbm.at[pl.ds(off, CH)],
                                          sws[b])
                    w2 = pltpu.async_copy(rrow[b], rout_hbm.at[pl.ds(off, CH)],
                                          sws[b])
                    load_idx(off + 2 * CH, b)
                    w1.wait()
                    w2.wait()
                    fire_gather(b)

            for b in (0, 1):
                off = base + (nch - 2 + b) * CH
                wait_gather(b)
                pltpu.sync_copy(srow[b], sout_hbm.at[pl.ds(off, CH)])
                pltpu.sync_copy(rrow[b], rout_hbm.at[pl.ds(off, CH)])

            if nextra:
                @pl.when(wid < nextra)
                def _():
                    eoff = NW * nch * CH + wid * CH
                    load_idx(eoff, 0)
                    fire_gather(0)
                    wait_gather(0)
                    pltpu.sync_copy(sr0, sout_hbm.at[pl.ds(eoff, CH)])
                    pltpu.sync_copy(rr0, rout_hbm.at[pl.ds(eoff, CH)])

        return gather

    gathers = tuple(make_gather(n, x)
                    for n, x in zip(SLICE_NCH, SLICE_EXTRA))

    @functools.partial(
        pl.kernel,
        out_type=jax.ShapeDtypeStruct((NC, A, H), jnp.float32),
        mesh=mesh,
        scratch_types=[
            pltpu.VMEM((2, CH), jnp.int32),
            pltpu.VMEM((CH, H), jnp.float32),
            pltpu.VMEM((CH, H), jnp.float32),
            pltpu.VMEM_SHARED((A, H), jnp.float32),
            pltpu.SemaphoreType.DMA,
            pltpu.SemaphoreType.DMA,
            pltpu.SemaphoreType.DMA,
            pltpu.SemaphoreType.DMA,
        ],
    )
    def scatter(m0, m1, m2, m3, r0_, r1_, r2_, r3_, zero_hbm, out_hbm,
                idx_v, mr0, mr1, acc_sh, sl0, sl1, sa0, sa1):
        c = lax.axis_index("c")
        s = lax.axis_index("s")
        wid = s * NC + c
        row0 = s * RPS
        mrow = (mr0, mr1)
        sls = (sl0, sl1)
        sas = (sa0, sa1)
        pltpu.sync_copy(zero_hbm.at[pl.ds(row0, RPS)],
                        acc_sh.at[pl.ds(row0, RPS)])

        @pl.when(s == 0)
        def _():
            pltpu.sync_copy(zero_hbm.at[pl.ds(TAIL0, TAILN)],
                            acc_sh.at[pl.ds(TAIL0, TAILN)])

        plsc.subcore_barrier()

        for (msg_hbm, recv_hbm, nch, nextra) in zip(
                (m0, m1, m2, m3), (r0_, r1_, r2_, r3_),
                SLICE_NCH, SLICE_EXTRA):
            base = wid * nch * CH

            def fire_load(off, b):
                pltpu.async_copy(recv_hbm.at[pl.ds(off, CH)], idx_v.at[b],
                                 sls[b])
                pltpu.async_copy(msg_hbm.at[pl.ds(off, CH)], mrow[b], sls[b])

            def wait_load(off, b):
                pltpu.make_async_copy(recv_hbm.at[pl.ds(off, CH)],
                                      idx_v.at[b], sls[b]).wait()
                pltpu.make_async_copy(msg_hbm.at[pl.ds(off, CH)],
                                      mrow[b], sls[b]).wait()

            fire_load(base, 0)
            fire_load(base + CH, 1)

            @pl.loop(0, (nch - 2) // 2)
            def _(j):
                for b in (0, 1):
                    off = base + (2 * j + b) * CH
                    wait_load(off, b)
                    a = pltpu.async_copy(mrow[b], acc_sh.at[idx_v.at[b]],
                                         sas[b], add=True)
                    a.wait()
                    fire_load(off + 2 * CH, b)

            for b in (0, 1):
                wait_load(base + (nch - 2 + b) * CH, b)
                pltpu.sync_copy(mrow[b], acc_sh.at[idx_v.at[b]], add=True)

            if nextra:
                @pl.when(wid < nextra)
                def _():
                    eoff = NW * nch * CH + wid * CH
                    pltpu.sync_copy(recv_hbm.at[pl.ds(eoff, CH)], idx_v.at[0])
                    pltpu.sync_copy(msg_hbm.at[pl.ds(eoff, CH)], mr0)
                    pltpu.sync_copy(mr0, acc_sh.at[idx_v.at[0]], add=True)

        plsc.subcore_barrier()
        pltpu.sync_copy(acc_sh.at[pl.ds(row0, RPS)],
                        out_hbm.at[c, pl.ds(row0, RPS)])

        @pl.when(s == 0)
        def _():
            pltpu.sync_copy(acc_sh.at[pl.ds(TAIL0, TAILN)],
                            out_hbm.at[c, pl.ds(TAIL0, TAILN)])

    return gathers, scatter


BE = 1280


def _mlp_body(se, re, p, w1a, w1b, b1, w2, b2, o):
    h = jnp.tanh(
        jnp.dot(se[...], w1a[...], preferred_element_type=jnp.float32)
        + jnp.dot(re[...], w1b[...], preferred_element_type=jnp.float32)
        + b1[...])
    m = jnp.tanh(jnp.dot(h, w2[...], preferred_element_type=jnp.float32) + b2[...])
    o[...] = m * p[...]


def _tc_mlp(send_emb, recv_emb, p, w1a, w1b, b1, w2, b2):
    es = send_emb.shape[0]
    return pl.pallas_call(
        _mlp_body,
        grid=(es // BE,),
        in_specs=[
            pl.BlockSpec((BE, H), lambda i: (i, 0)),
            pl.BlockSpec((BE, H), lambda i: (i, 0)),
            pl.BlockSpec((BE, 1), lambda i: (i, 0)),
            pl.BlockSpec((H, H), lambda i: (0, 0)),
            pl.BlockSpec((H, H), lambda i: (0, 0)),
            pl.BlockSpec((1, H), lambda i: (0, 0)),
            pl.BlockSpec((H, H), lambda i: (0, 0)),
            pl.BlockSpec((1, H), lambda i: (0, 0)),
        ],
        out_specs=pl.BlockSpec((BE, H), lambda i: (i, 0)),
        out_shape=jax.ShapeDtypeStruct((es, H), jnp.float32),
    )(send_emb, recv_emb, p, w1a, w1b, b1, w2, b2)


def _add_body(a, o):
    o[...] = a[0] + a[1]


def _tc_add(partials):
    return pl.pallas_call(
        _add_body,
        grid=(10,),
        in_specs=[pl.BlockSpec((NC, A // 10, H), lambda i: (0, i, 0))],
        out_specs=pl.BlockSpec((A // 10, H), lambda i: (i, 0)),
        out_shape=jax.ShapeDtypeStruct((A, H), jnp.float32),
    )(partials)


def kernel(node_embedding, edge_probs, send_edges, recv_edges, node_masks,
           W1, b1, W2, b2):
    del node_masks
    x = node_embedding[0]
    p = edge_probs[0, :, 1:2]
    gathers, scatter = _sc_kernels()

    w1a, w1b = W1[:H], W1[H:]
    b1r, b2r = b1.reshape(1, H), b2.reshape(1, H)

    msgs, recvs = [], []
    off = 0
    for g, es in zip(gathers, SLICE_ES):
        se_s = send_edges[off:off + es]
        re_s = recv_edges[off:off + es]
        p_s = p[off:off + es]
        send_emb, recv_emb = g(x, se_s, re_s)
        msgs.append(_tc_mlp(send_emb, recv_emb, p_s, w1a, w1b, b1r, W2, b2r))
        recvs.append(re_s)
        off += es

    zeros = jnp.zeros((A, H), jnp.float32)
    partials = scatter(*msgs, *recvs, zeros)
    return _tc_add(partials)[None]


# speedup vs baseline: 1.0135x; 1.0135x over previous
import functools

import jax
import jax.numpy as jnp
from jax import lax
from jax.experimental import pallas as pl
from jax.experimental.pallas import tpu as pltpu
from jax.experimental.pallas import tpu_sc as plsc

A, E, H = 10000, 320000, 128
NC, NS = 2, 16
NW = NC * NS
CH = 128
NCH = 78
EPW = NCH * CH
EMAIN = NW * EPW
NX = (E - EMAIN) // CH
RPS = 624
TAIL0 = NS * RPS
TAILN = A - TAIL0


@functools.cache
def _sc_kernels():
    mesh = plsc.VectorSubcoreMesh(core_axis_name="c", subcore_axis_name="s")

    @functools.partial(
        pl.kernel,
        out_type=(jax.ShapeDtypeStruct((E, H), jnp.float32),
                  jax.ShapeDtypeStruct((E, H), jnp.float32)),
        mesh=mesh,
        scratch_types=[
            pltpu.VMEM((NCH, CH), jnp.int32),
            pltpu.VMEM((NCH, CH), jnp.int32),
            pltpu.VMEM((1, CH), jnp.int32),
            pltpu.VMEM((1, CH), jnp.int32),
            pltpu.VMEM((CH, H), jnp.float32),
            pltpu.VMEM((CH, H), jnp.float32),
            pltpu.VMEM((CH, H), jnp.float32),
            pltpu.VMEM((CH, H), jnp.float32),
            pltpu.SemaphoreType.DMA,
            pltpu.SemaphoreType.DMA,
            pltpu.SemaphoreType.DMA,
            pltpu.SemaphoreType.DMA,
        ],
    )
    def _sc_gather(node_hbm, sm_hbm, rm_hbm, sx_hbm, rx_hbm,
                   sout_hbm, rout_hbm,
                   sidx_v, ridx_v, sxi_v, rxi_v, sr0, sr1, rr0, rr1,
                   sg0, sg1, sw0, sw1):
        c = lax.axis_index("c")
        s = lax.axis_index("s")
        wid = s * NC + c
        base = wid * EPW
        srow = (sr0, sr1)
        rrow = (rr0, rr1)
        sgs = (sg0, sg1)
        sws = (sw0, sw1)

        pltpu.sync_copy(sm_hbm.at[wid], sidx_v)
        pltpu.sync_copy(rm_hbm.at[wid], ridx_v)

        def fire_gather(i, b):
            pltpu.async_copy(node_hbm.at[sidx_v.at[i]], srow[b], sgs[b])
            pltpu.async_copy(node_hbm.at[ridx_v.at[i]], rrow[b], sgs[b])

        def wait_gather(i, b):
            pltpu.make_async_copy(node_hbm.at[sidx_v.at[i]], srow[b],
                                  sgs[b]).wait()
            pltpu.make_async_copy(node_hbm.at[ridx_v.at[i]], rrow[b],
                                  sgs[b]).wait()

        fire_gather(0, 0)
        fire_gather(1, 1)

        @pl.loop(0, (NCH - 2) // 2)
        def _(j):
            for b in (0, 1):
                i = 2 * j + b
                off = base + i * CH
                wait_gather(i, b)
                w1 = pltpu.async_copy(srow[b], sout_hbm.at[pl.ds(off, CH)],
                                      sws[b])
                w2 = pltpu.async_copy(rrow[b], rout_hbm.at[pl.ds(off, CH)],
                                      sws[b])
                w1.wait()
                w2.wait()
                fire_gather(i + 2, b)

        for b in (0, 1):
            i = NCH - 2 + b
            off = base + i * CH
            wait_gather(i, b)
            pltpu.sync_copy(srow[b], sout_hbm.at[pl.ds(off, CH)])
            pltpu.sync_copy(rrow[b], rout_hbm.at[pl.ds(off, CH)])

        @pl.when(wid < NX)
        def _():
            xoff = EMAIN + wid * CH
            pltpu.sync_copy(sx_hbm.at[wid], sxi_v.at[0])
            pltpu.sync_copy(rx_hbm.at[wid], rxi_v.at[0])
            g1 = pltpu.async_copy(node_hbm.at[sxi_v.at[0]], sr0, sg0)
            g2 = pltpu.async_copy(node_hbm.at[rxi_v.at[0]], rr0, sg0)
            g1.wait()
            g2.wait()
            pltpu.sync_copy(sr0, sout_hbm.at[pl.ds(xoff, CH)])
            pltpu.sync_copy(rr0, rout_hbm.at[pl.ds(xoff, CH)])

    @functools.partial(
        pl.kernel,
        out_type=jax.ShapeDtypeStruct((NC, A, H), jnp.float32),
        mesh=mesh,
        scratch_types=[
            pltpu.VMEM((NCH, CH), jnp.int32),
            pltpu.VMEM((1, CH), jnp.int32),
            pltpu.VMEM((CH, H), jnp.float32),
            pltpu.VMEM((CH, H), jnp.float32),
            pltpu.VMEM_SHARED((A, H), jnp.float32),
            pltpu.SemaphoreType.DMA,
            pltpu.SemaphoreType.DMA,
            pltpu.SemaphoreType.DMA,
            pltpu.SemaphoreType.DMA,
        ],
    )
    def _sc_scatter(msg_hbm, rm_hbm, rx_hbm, zero_hbm, out_hbm,
                    ridx_v, rxi_v, mr0, mr1, acc_sh, sl0, sl1, sa0, sa1):
        c = lax.axis_index("c")
        s = lax.axis_index("s")
        wid = s * NC + c
        base = wid * EPW
        r0 = s * RPS
        mrow = (mr0, mr1)
        sls = (sl0, sl1)
        sas = (sa0, sa1)
        pltpu.sync_copy(zero_hbm.at[pl.ds(r0, RPS)], acc_sh.at[pl.ds(r0, RPS)])

        @pl.when(s == 0)
        def _():
            pltpu.sync_copy(zero_hbm.at[pl.ds(TAIL0, TAILN)],
                            acc_sh.at[pl.ds(TAIL0, TAILN)])

        pltpu.sync_copy(rm_hbm.at[wid], ridx_v)
        plsc.subcore_barrier()

        def fire_load(i, b):
            off = base + i * CH
            pltpu.async_copy(msg_hbm.at[pl.ds(off, CH)], mrow[b], sls[b])

        def wait_load(i, b):
            off = base + i * CH
            pltpu.make_async_copy(msg_hbm.at[pl.ds(off, CH)], mrow[b],
                                  sls[b]).wait()

        fire_load(0, 0)
        fire_load(1, 1)

        @pl.loop(0, (NCH - 2) // 2)
        def _(j):
            for b in (0, 1):
                i = 2 * j + b
                wait_load(i, b)
                a = pltpu.async_copy(mrow[b], acc_sh.at[ridx_v.at[i]],
                                     sas[b], add=True)
                a.wait()
                fire_load(i + 2, b)

        for b in (0, 1):
            i = NCH - 2 + b
            wait_load(i, b)
            pltpu.sync_copy(mrow[b], acc_sh.at[ridx_v.at[i]], add=True)

        @pl.when(wid < NX)
        def _():
            xoff = EMAIN + wid * CH
            pltpu.sync_copy(rx_hbm.at[wid], rxi_v.at[0])
            pltpu.sync_copy(msg_hbm.at[pl.ds(xoff, CH)], mr0)
            pltpu.sync_copy(mr0, acc_sh.at[rxi_v.at[0]], add=True)

        plsc.subcore_barrier()
        pltpu.sync_copy(acc_sh.at[pl.ds(r0, RPS)], out_hbm.at[c, pl.ds(r0, RPS)])

        @pl.when(s == 0)
        def _():
            pltpu.sync_copy(acc_sh.at[pl.ds(TAIL0, TAILN)],
                            out_hbm.at[c, pl.ds(TAIL0, TAILN)])

    return _sc_gather, _sc_scatter


BE = 2000


def _mlp_body(sb, rb, p, w1a, w1b, b1, w2, b2, o):
    f32 = jnp.float32
    bh = jnp.bfloat16
    z = (jnp.dot(sb[...].astype(bh), w1a[...], preferred_element_type=f32)
         + jnp.dot(rb[...].astype(bh), w1b[...], preferred_element_type=f32)
         + b1[...])
    h = jnp.tanh(z).astype(bh)
    m = jnp.tanh(jnp.dot(h, w2[...], preferred_element_type=f32) + b2[...])
    o[...] = m * p[...]


def _tc_mlp(sb, rb, p, w1a, w1b, b1, w2, b2):
    return pl.pallas_call(
        _mlp_body,
        grid=(E // BE,),
        in_specs=[
            pl.BlockSpec((BE, H), lambda i: (i, 0)),
            pl.BlockSpec((BE, H), lambda i: (i, 0)),
            pl.BlockSpec((BE, 1), lambda i: (i, 0)),
            pl.BlockSpec((H, H), lambda i: (0, 0)),
            pl.BlockSpec((H, H), lambda i: (0, 0)),
            pl.BlockSpec((1, H), lambda i: (0, 0)),
            pl.BlockSpec((H, H), lambda i: (0, 0)),
            pl.BlockSpec((1, H), lambda i: (0, 0)),
        ],
        out_specs=pl.BlockSpec((BE, H), lambda i: (i, 0)),
        out_shape=jax.ShapeDtypeStruct((E, H), jnp.float32),
    )(sb, rb, p, w1a, w1b, b1, w2, b2)


def _add_body(a, o):
    o[...] = a[0] + a[1]


def _tc_add(partials):
    return pl.pallas_call(
        _add_body,
        grid=(10,),
        in_specs=[pl.BlockSpec((NC, A // 10, H), lambda i: (0, i, 0))],
        out_specs=pl.BlockSpec((A // 10, H), lambda i: (i, 0)),
        out_shape=jax.ShapeDtypeStruct((A, H), jnp.float32),
    )(partials)


def kernel(node_embedding, edge_probs, send_edges, recv_edges, node_masks,
           W1, b1, W2, b2):
    del node_masks
    x = node_embedding[0]
    p = edge_probs[0, :, 1:2]

    sm = send_edges[:EMAIN].reshape(NW, NCH, CH)
    rm = recv_edges[:EMAIN].reshape(NW, NCH, CH)
    sx = send_edges[EMAIN:].reshape(NX, CH)
    rx = recv_edges[EMAIN:].reshape(NX, CH)

    _sc_gather, _sc_scatter = _sc_kernels()
    sb, rb = _sc_gather(x, sm, rm, sx, rx)

    bh = jnp.bfloat16
    msg = _tc_mlp(sb, rb, p,
                  W1[:H].astype(bh), W1[H:].astype(bh),
                  b1.reshape(1, H), W2.astype(bh), b2.reshape(1, H))
    zeros = jnp.zeros((A, H), jnp.float32)
    partials = _sc_scatter(msg, rm, rx, zeros)
    return _tc_add(partials)[None]


# speedup vs baseline: 1.0423x; 1.0284x over previous
import functools

import jax
import jax.numpy as jnp
from jax import lax
from jax.experimental import pallas as pl
from jax.experimental.pallas import tpu as pltpu
from jax.experimental.pallas import tpu_sc as plsc

A, E, H = 10000, 320000, 128
NC, NS = 2, 16
NW = NC * NS
CH = 128
NCH = 78
EPW = NCH * CH
EMAIN = NW * EPW
NX = (E - EMAIN) // CH
RPS = 624
TAIL0 = NS * RPS
TAILN = A - TAIL0


@functools.cache
def _sc_kernels():
    mesh = plsc.VectorSubcoreMesh(core_axis_name="c", subcore_axis_name="s")

    @functools.partial(
        pl.kernel,
        out_type=(jax.ShapeDtypeStruct((E, H), jnp.float32),
                  jax.ShapeDtypeStruct((E, H), jnp.float32)),
        mesh=mesh,
        scratch_types=[
            pltpu.VMEM((NCH, CH), jnp.int32),
            pltpu.VMEM((NCH, CH), jnp.int32),
            pltpu.VMEM((1, CH), jnp.int32),
            pltpu.VMEM((1, CH), jnp.int32),
            pltpu.VMEM((CH, H), jnp.float32),
            pltpu.VMEM((CH, H), jnp.float32),
            pltpu.VMEM((CH, H), jnp.float32),
            pltpu.VMEM((CH, H), jnp.float32),
            pltpu.SemaphoreType.DMA,
            pltpu.SemaphoreType.DMA,
            pltpu.SemaphoreType.DMA,
            pltpu.SemaphoreType.DMA,
        ],
    )
    def _sc_gather(node_hbm, sm_hbm, rm_hbm, sx_hbm, rx_hbm,
                   sout_hbm, rout_hbm,
                   sidx_v, ridx_v, sxi_v, rxi_v, sr0, sr1, rr0, rr1,
                   sg0, sg1, sw0, sw1):
        c = lax.axis_index("c")
        s = lax.axis_index("s")
        wid = s * NC + c
        base = wid * EPW
        srow = (sr0, sr1)
        rrow = (rr0, rr1)
        sgs = (sg0, sg1)
        sws = (sw0, sw1)

        pltpu.sync_copy(sm_hbm.at[wid], sidx_v)
        pltpu.sync_copy(rm_hbm.at[wid], ridx_v)

        def fire_gather(i, b):
            pltpu.async_copy(node_hbm.at[sidx_v.at[i]], srow[b], sgs[b])
            pltpu.async_copy(node_hbm.at[ridx_v.at[i]], rrow[b], sgs[b])

        def wait_gather(i, b):
            pltpu.make_async_copy(node_hbm.at[sidx_v.at[i]], srow[b],
                                  sgs[b]).wait()
            pltpu.make_async_copy(node_hbm.at[ridx_v.at[i]], rrow[b],
                                  sgs[b]).wait()

        fire_gather(0, 0)
        fire_gather(1, 1)

        def fire_write(i, b):
            off = base + i * CH
            pltpu.async_copy(srow[b], sout_hbm.at[pl.ds(off, CH)], sws[b])
            pltpu.async_copy(rrow[b], rout_hbm.at[pl.ds(off, CH)], sws[b])

        def wait_write(i, b):
            off = base + i * CH
            pltpu.make_async_copy(srow[b], sout_hbm.at[pl.ds(off, CH)],
                                  sws[b]).wait()
            pltpu.make_async_copy(rrow[b], rout_hbm.at[pl.ds(off, CH)],
                                  sws[b]).wait()

        @pl.loop(0, (NCH - 2) // 2)
        def _(j):
            i = 2 * j
            wait_gather(i, 0)
            fire_write(i, 0)
            wait_gather(i + 1, 1)
            fire_write(i + 1, 1)
            wait_write(i, 0)
            fire_gather(i + 2, 0)
            wait_write(i + 1, 1)
            fire_gather(i + 3, 1)

        for b in (0, 1):
            i = NCH - 2 + b
            off = base + i * CH
            wait_gather(i, b)
            pltpu.sync_copy(srow[b], sout_hbm.at[pl.ds(off, CH)])
            pltpu.sync_copy(rrow[b], rout_hbm.at[pl.ds(off, CH)])

        @pl.when(wid < NX)
        def _():
            xoff = EMAIN + wid * CH
            pltpu.sync_copy(sx_hbm.at[wid], sxi_v.at[0])
            pltpu.sync_copy(rx_hbm.at[wid], rxi_v.at[0])
            g1 = pltpu.async_copy(node_hbm.at[sxi_v.at[0]], sr0, sg0)
            g2 = pltpu.async_copy(node_hbm.at[rxi_v.at[0]], rr0, sg0)
            g1.wait()
            g2.wait()
            pltpu.sync_copy(sr0, sout_hbm.at[pl.ds(xoff, CH)])
            pltpu.sync_copy(rr0, rout_hbm.at[pl.ds(xoff, CH)])

    @functools.partial(
        pl.kernel,
        out_type=jax.ShapeDtypeStruct((NC, A, H), jnp.float32),
        mesh=mesh,
        scratch_types=[
            pltpu.VMEM((NCH, CH), jnp.int32),
            pltpu.VMEM((1, CH), jnp.int32),
            pltpu.VMEM((CH, H), jnp.float32),
            pltpu.VMEM((CH, H), jnp.float32),
            pltpu.VMEM_SHARED((A, H), jnp.float32),
            pltpu.SemaphoreType.DMA,
            pltpu.SemaphoreType.DMA,
            pltpu.SemaphoreType.DMA,
            pltpu.SemaphoreType.DMA,
        ],
    )
    def _sc_scatter(msg_hbm, rm_hbm, rx_hbm, zero_hbm, out_hbm,
                    ridx_v, rxi_v, mr0, mr1, acc_sh, sl0, sl1, sa0, sa1):
        c = lax.axis_index("c")
        s = lax.axis_index("s")
        wid = s * NC + c
        base = wid * EPW
        r0 = s * RPS
        mrow = (mr0, mr1)
        sls = (sl0, sl1)
        sas = (sa0, sa1)
        pltpu.sync_copy(zero_hbm.at[pl.ds(r0, RPS)], acc_sh.at[pl.ds(r0, RPS)])

        @pl.when(s == 0)
        def _():
            pltpu.sync_copy(zero_hbm.at[pl.ds(TAIL0, TAILN)],
                            acc_sh.at[pl.ds(TAIL0, TAILN)])

        pltpu.sync_copy(rm_hbm.at[wid], ridx_v)
        plsc.subcore_barrier()

        def fire_load(i, b):
            off = base + i * CH
            pltpu.async_copy(msg_hbm.at[pl.ds(off, CH)], mrow[b], sls[b])

        def wait_load(i, b):
            off = base + i * CH
            pltpu.make_async_copy(msg_hbm.at[pl.ds(off, CH)], mrow[b],
                                  sls[b]).wait()

        fire_load(0, 0)
        fire_load(1, 1)

        @pl.loop(0, (NCH - 2) // 2)
        def _(j):
            for b in (0, 1):
                i = 2 * j + b
                wait_load(i, b)
                a = pltpu.async_copy(mrow[b], acc_sh.at[ridx_v.at[i]],
                                     sas[b], add=True)
                a.wait()
                fire_load(i + 2, b)

        for b in (0, 1):
            i = NCH - 2 + b
            wait_load(i, b)
            pltpu.sync_copy(mrow[b], acc_sh.at[ridx_v.at[i]], add=True)

        @pl.when(wid < NX)
        def _():
            xoff = EMAIN + wid * CH
            pltpu.sync_copy(rx_hbm.at[wid], rxi_v.at[0])
            pltpu.sync_copy(msg_hbm.at[pl.ds(xoff, CH)], mr0)
            pltpu.sync_copy(mr0, acc_sh.at[rxi_v.at[0]], add=True)

        plsc.subcore_barrier()
        pltpu.sync_copy(acc_sh.at[pl.ds(r0, RPS)], out_hbm.at[c, pl.ds(r0, RPS)])

        @pl.when(s == 0)
        def _():
            pltpu.sync_copy(acc_sh.at[pl.ds(TAIL0, TAILN)],
                            out_hbm.at[c, pl.ds(TAIL0, TAILN)])

    return _sc_gather, _sc_scatter


BE = 2000


def _mlp_body(sb, rb, p, w1a, w1b, b1, w2, b2, o):
    f32 = jnp.float32
    z = (jnp.dot(sb[...], w1a[...], preferred_element_type=f32)
         + jnp.dot(rb[...], w1b[...], preferred_element_type=f32)
         + b1[...])
    h = jnp.tanh(z)
    m = jnp.tanh(jnp.dot(h, w2[...], preferred_element_type=f32) + b2[...])
    o[...] = m * p[...]


def _tc_mlp(sb, rb, p, w1a, w1b, b1, w2, b2):
    return pl.pallas_call(
        _mlp_body,
        grid=(E // BE,),
        in_specs=[
            pl.BlockSpec((BE, H), lambda i: (i, 0)),
            pl.BlockSpec((BE, H), lambda i: (i, 0)),
            pl.BlockSpec((BE, 1), lambda i: (i, 0)),
            pl.BlockSpec((H, H), lambda i: (0, 0)),
            pl.BlockSpec((H, H), lambda i: (0, 0)),
            pl.BlockSpec((1, H), lambda i: (0, 0)),
            pl.BlockSpec((H, H), lambda i: (0, 0)),
            pl.BlockSpec((1, H), lambda i: (0, 0)),
        ],
        out_specs=pl.BlockSpec((BE, H), lambda i: (i, 0)),
        out_shape=jax.ShapeDtypeStruct((E, H), jnp.float32),
        compiler_params=pltpu.CompilerParams(
            dimension_semantics=("parallel",)),
    )(sb, rb, p, w1a, w1b, b1, w2, b2)


def _add_body(a, o):
    o[...] = a[0] + a[1]


def _tc_add(partials):
    return pl.pallas_call(
        _add_body,
        grid=(10,),
        in_specs=[pl.BlockSpec((NC, A // 10, H), lambda i: (0, i, 0))],
        out_specs=pl.BlockSpec((A // 10, H), lambda i: (i, 0)),
        out_shape=jax.ShapeDtypeStruct((A, H), jnp.float32),
        compiler_params=pltpu.CompilerParams(
            dimension_semantics=("parallel",)),
    )(partials)


def kernel(node_embedding, edge_probs, send_edges, recv_edges, node_masks,
           W1, b1, W2, b2):
    del node_masks
    x = node_embedding[0]
    p = edge_probs[0, :, 1:2]

    sm = send_edges[:EMAIN].reshape(NW, NCH, CH)
    rm = recv_edges[:EMAIN].reshape(NW, NCH, CH)
    sx = send_edges[EMAIN:].reshape(NX, CH)
    rx = recv_edges[EMAIN:].reshape(NX, CH)

    _sc_gather, _sc_scatter = _sc_kernels()
    sb, rb = _sc_gather(x, sm, rm, sx, rx)

    bh = jnp.bfloat16
    msg = _tc_mlp(sb, rb, p,
                  W1[:H].astype(bh), W1[H:].astype(bh),
                  b1.reshape(1, H), W2.astype(bh), b2.reshape(1, H))
    zeros = jnp.zeros((A, H), jnp.float32)
    partials = _sc_scatter(msg, rm, rx, zeros)
    return _tc_add(partials)[None]


# speedup vs baseline: 1.1310x; 1.0851x over previous
import functools

import jax
import jax.numpy as jnp
from jax import lax
from jax.experimental import pallas as pl
from jax.experimental.pallas import tpu as pltpu
from jax.experimental.pallas import tpu_sc as plsc

A, E, H = 10000, 320000, 128
NC, NS = 2, 16
NW = NC * NS
CH = 128
NCH = 78
EPW = NCH * CH
EMAIN = NW * EPW
NX = (E - EMAIN) // CH
RPS = 624
TAIL0 = NS * RPS
TAILN = A - TAIL0


@functools.cache
def _sc_kernels():
    mesh = plsc.VectorSubcoreMesh(core_axis_name="c", subcore_axis_name="s")

    @functools.partial(
        pl.kernel,
        out_type=(jax.ShapeDtypeStruct((E, H), jnp.float32),
                  jax.ShapeDtypeStruct((E, H), jnp.float32)),
        mesh=mesh,
        scratch_types=[
            pltpu.VMEM((NCH, CH), jnp.int32),
            pltpu.VMEM((NCH, CH), jnp.int32),
            pltpu.VMEM((1, CH), jnp.int32),
            pltpu.VMEM((1, CH), jnp.int32),
            pltpu.VMEM((CH, H), jnp.float32),
            pltpu.VMEM((CH, H), jnp.float32),
            pltpu.VMEM((CH, H), jnp.float32),
            pltpu.VMEM((CH, H), jnp.float32),
            pltpu.SemaphoreType.DMA,
            pltpu.SemaphoreType.DMA,
            pltpu.SemaphoreType.DMA,
            pltpu.SemaphoreType.DMA,
        ],
    )
    def _sc_gather(node_hbm, sm_hbm, rm_hbm, sx_hbm, rx_hbm,
                   sout_hbm, rout_hbm,
                   sidx_v, ridx_v, sxi_v, rxi_v, sr0, sr1, rr0, rr1,
                   sg0, sg1, sw0, sw1):
        c = lax.axis_index("c")
        s = lax.axis_index("s")
        wid = s * NC + c
        base = wid * EPW
        srow = (sr0, sr1)
        rrow = (rr0, rr1)
        sgs = (sg0, sg1)
        sws = (sw0, sw1)

        pltpu.sync_copy(sm_hbm.at[wid], sidx_v)
        pltpu.sync_copy(rm_hbm.at[wid], ridx_v)

        def fire_gather(i, b):
            pltpu.async_copy(node_hbm.at[sidx_v.at[i]], srow[b], sgs[b])
            pltpu.async_copy(node_hbm.at[ridx_v.at[i]], rrow[b], sgs[b])

        def wait_gather(i, b):
            pltpu.make_async_copy(node_hbm.at[sidx_v.at[i]], srow[b],
                                  sgs[b]).wait()
            pltpu.make_async_copy(node_hbm.at[ridx_v.at[i]], rrow[b],
                                  sgs[b]).wait()

        fire_gather(0, 0)
        fire_gather(1, 1)

        @pl.loop(0, (NCH - 2) // 2)
        def _(j):
            for b in (0, 1):
                i = 2 * j + b
                off = base + i * CH
                wait_gather(i, b)
                w1 = pltpu.async_copy(srow[b], sout_hbm.at[pl.ds(off, CH)],
                                      sws[b])
                w2 = pltpu.async_copy(rrow[b], rout_hbm.at[pl.ds(off, CH)],
                                      sws[b])
                w1.wait()
                w2.wait()
                fire_gather(i + 2, b)

        for b in (0, 1):
            i = NCH - 2 + b
            off = base + i * CH
            wait_gather(i, b)
            pltpu.sync_copy(srow[b], sout_hbm.at[pl.ds(off, CH)])
            pltpu.sync_copy(rrow[b], rout_hbm.at[pl.ds(off, CH)])

        @pl.when(wid < NX)
        def _():
            xoff = EMAIN + wid * CH
            pltpu.sync_copy(sx_hbm.at[wid], sxi_v.at[0])
            pltpu.sync_copy(rx_hbm.at[wid], rxi_v.at[0])
            g1 = pltpu.async_copy(node_hbm.at[sxi_v.at[0]], sr0, sg0)
            g2 = pltpu.async_copy(node_hbm.at[rxi_v.at[0]], rr0, sg0)
            g1.wait()
            g2.wait()
            pltpu.sync_copy(sr0, sout_hbm.at[pl.ds(xoff, CH)])
            pltpu.sync_copy(rr0, rout_hbm.at[pl.ds(xoff, CH)])

    @functools.partial(
        pl.kernel,
        out_type=jax.ShapeDtypeStruct((NC, A, H), jnp.float32),
        mesh=mesh,
        scratch_types=[
            pltpu.VMEM((NCH, CH), jnp.int32),
            pltpu.VMEM((1, CH), jnp.int32),
            pltpu.VMEM((CH, H), jnp.float32),
            pltpu.VMEM((CH, H), jnp.float32),
            pltpu.VMEM_SHARED((A, H), jnp.float32),
            pltpu.SemaphoreType.DMA,
            pltpu.SemaphoreType.DMA,
            pltpu.SemaphoreType.DMA,
            pltpu.SemaphoreType.DMA,
        ],
    )
    def _sc_scatter(msg_hbm, rm_hbm, rx_hbm, zero_hbm, out_hbm,
                    ridx_v, rxi_v, mr0, mr1, acc_sh, sl0, sl1, sa0, sa1):
        c = lax.axis_index("c")
        s = lax.axis_index("s")
        wid = s * NC + c
        base = wid * EPW
        r0 = s * RPS
        mrow = (mr0, mr1)
        sls = (sl0, sl1)
        sas = (sa0, sa1)
        pltpu.sync_copy(zero_hbm.at[pl.ds(r0, RPS)], acc_sh.at[pl.ds(r0, RPS)])

        @pl.when(s == 0)
        def _():
            pltpu.sync_copy(zero_hbm.at[pl.ds(TAIL0, TAILN)],
                            acc_sh.at[pl.ds(TAIL0, TAILN)])

        pltpu.sync_copy(rm_hbm.at[wid], ridx_v)
        plsc.subcore_barrier()

        def fire_load(i, b):
            off = base + i * CH
            pltpu.async_copy(msg_hbm.at[pl.ds(off, CH)], mrow[b], sls[b])

        def wait_load(i, b):
            off = base + i * CH
            pltpu.make_async_copy(msg_hbm.at[pl.ds(off, CH)], mrow[b],
                                  sls[b]).wait()

        fire_load(0, 0)
        fire_load(1, 1)

        @pl.loop(0, (NCH - 2) // 2)
        def _(j):
            for b in (0, 1):
                i = 2 * j + b
                wait_load(i, b)
                a = pltpu.async_copy(mrow[b], acc_sh.at[ridx_v.at[i]],
                                     sas[b], add=True)
                a.wait()
                fire_load(i + 2, b)

        for b in (0, 1):
            i = NCH - 2 + b
            wait_load(i, b)
            pltpu.sync_copy(mrow[b], acc_sh.at[ridx_v.at[i]], add=True)

        @pl.when(wid < NX)
        def _():
            xoff = EMAIN + wid * CH
            pltpu.sync_copy(rx_hbm.at[wid], rxi_v.at[0])
            pltpu.sync_copy(msg_hbm.at[pl.ds(xoff, CH)], mr0)
            pltpu.sync_copy(mr0, acc_sh.at[rxi_v.at[0]], add=True)

        plsc.subcore_barrier()
        pltpu.sync_copy(acc_sh.at[pl.ds(r0, RPS)], out_hbm.at[c, pl.ds(r0, RPS)])

        @pl.when(s == 0)
        def _():
            pltpu.sync_copy(acc_sh.at[pl.ds(TAIL0, TAILN)],
                            out_hbm.at[c, pl.ds(TAIL0, TAILN)])

    return _sc_gather, _sc_scatter


BE = 4000


def _mlp_body(sb, rb, p, w1a, w1b, b1, w2, b2, o):
    f32 = jnp.float32
    z = (jnp.dot(sb[...], w1a[...], preferred_element_type=f32)
         + jnp.dot(rb[...], w1b[...], preferred_element_type=f32)
         + b1[...])
    h = jnp.tanh(z)
    m = jnp.tanh(jnp.dot(h, w2[...], preferred_element_type=f32) + b2[...])
    o[...] = m * p[...]


def _tc_mlp(sb, rb, p, w1a, w1b, b1, w2, b2):
    return pl.pallas_call(
        _mlp_body,
        grid=(E // BE,),
        in_specs=[
            pl.BlockSpec((BE, H), lambda i: (i, 0)),
            pl.BlockSpec((BE, H), lambda i: (i, 0)),
            pl.BlockSpec((BE, 1), lambda i: (i, 0)),
            pl.BlockSpec((H, H), lambda i: (0, 0)),
            pl.BlockSpec((H, H), lambda i: (0, 0)),
            pl.BlockSpec((1, H), lambda i: (0, 0)),
            pl.BlockSpec((H, H), lambda i: (0, 0)),
            pl.BlockSpec((1, H), lambda i: (0, 0)),
        ],
        out_specs=pl.BlockSpec((BE, H), lambda i: (i, 0)),
        out_shape=jax.ShapeDtypeStruct((E, H), jnp.float32),
    )(sb, rb, p, w1a, w1b, b1, w2, b2)


def _add_body(a, o):
    o[...] = a[0] + a[1]


def _tc_add(partials):
    return pl.pallas_call(
        _add_body,
        grid=(10,),
        in_specs=[pl.BlockSpec((NC, A // 10, H), lambda i: (0, i, 0))],
        out_specs=pl.BlockSpec((A // 10, H), lambda i: (i, 0)),
        out_shape=jax.ShapeDtypeStruct((A, H), jnp.float32),
    )(partials)


def kernel(node_embedding, edge_probs, send_edges, recv_edges, node_masks,
           W1, b1, W2, b2):
    del node_masks
    x = node_embedding[0]
    p = edge_probs[0, :, 1:2]

    sm = send_edges[:EMAIN].reshape(NW, NCH, CH)
    rm = recv_edges[:EMAIN].reshape(NW, NCH, CH)
    sx = send_edges[EMAIN:].reshape(NX, CH)
    rx = recv_edges[EMAIN:].reshape(NX, CH)

    _sc_gather, _sc_scatter = _sc_kernels()
    sb, rb = _sc_gather(x, sm, rm, sx, rx)

    bh = jnp.bfloat16
    msg = _tc_mlp(sb, rb, p,
                  W1[:H].astype(bh), W1[H:].astype(bh),
                  b1.reshape(1, H), W2.astype(bh), b2.reshape(1, H))
    zeros = jnp.zeros((A, H), jnp.float32)
    partials = _sc_scatter(msg, rm, rx, zeros)
    return _tc_add(partials)[None]


# speedup vs baseline: 1.1421x; 1.0098x over previous
import functools

import jax
import jax.numpy as jnp
from jax import lax
from jax.experimental import pallas as pl
from jax.experimental.pallas import tpu as pltpu
from jax.experimental.pallas import tpu_sc as plsc

A, E, H = 10000, 320000, 128
NC, NS = 2, 16
NW = NC * NS
CH = 128
NCH = 78
EPW = NCH * CH
EMAIN = NW * EPW
NX = (E - EMAIN) // CH
RPS = 624
TAIL0 = NS * RPS
TAILN = A - TAIL0


@functools.cache
def _sc_kernels():
    mesh = plsc.VectorSubcoreMesh(core_axis_name="c", subcore_axis_name="s")

    @functools.partial(
        pl.kernel,
        out_type=(jax.ShapeDtypeStruct((E, H), jnp.float32),
                  jax.ShapeDtypeStruct((E, H), jnp.float32)),
        mesh=mesh,
        scratch_types=[
            pltpu.VMEM((NCH, CH), jnp.int32),
            pltpu.VMEM((NCH, CH), jnp.int32),
            pltpu.VMEM((1, CH), jnp.int32),
            pltpu.VMEM((1, CH), jnp.int32),
            pltpu.VMEM((CH, H), jnp.float32),
            pltpu.VMEM((CH, H), jnp.float32),
            pltpu.VMEM((CH, H), jnp.float32),
            pltpu.VMEM((CH, H), jnp.float32),
            pltpu.SemaphoreType.DMA,
            pltpu.SemaphoreType.DMA,
            pltpu.SemaphoreType.DMA,
            pltpu.SemaphoreType.DMA,
        ],
    )
    def _sc_gather(node_hbm, sm_hbm, rm_hbm, sx_hbm, rx_hbm,
                   sout_hbm, rout_hbm,
                   sidx_v, ridx_v, sxi_v, rxi_v, sr0, sr1, rr0, rr1,
                   sg0, sg1, sw0, sw1):
        c = lax.axis_index("c")
        s = lax.axis_index("s")
        wid = s * NC + c
        base = wid * EPW
        srow = (sr0, sr1)
        rrow = (rr0, rr1)
        sgs = (sg0, sg1)
        sws = (sw0, sw1)

        pltpu.sync_copy(sm_hbm.at[wid], sidx_v)
        pltpu.sync_copy(rm_hbm.at[wid], ridx_v)

        def fire_gather(i, b):
            pltpu.async_copy(node_hbm.at[sidx_v.at[i]], srow[b], sgs[b])
            pltpu.async_copy(node_hbm.at[ridx_v.at[i]], rrow[b], sgs[b])

        def wait_gather(i, b):
            pltpu.make_async_copy(node_hbm.at[sidx_v.at[i]], srow[b],
                                  sgs[b]).wait()
            pltpu.make_async_copy(node_hbm.at[ridx_v.at[i]], rrow[b],
                                  sgs[b]).wait()

        fire_gather(0, 0)
        fire_gather(1, 1)

        @pl.loop(0, (NCH - 2) // 2)
        def _(j):
            for b in (0, 1):
                i = 2 * j + b
                off = base + i * CH
                wait_gather(i, b)
                w1 = pltpu.async_copy(srow[b], sout_hbm.at[pl.ds(off, CH)],
                                      sws[b])
                w2 = pltpu.async_copy(rrow[b], rout_hbm.at[pl.ds(off, CH)],
                                      sws[b])
                w1.wait()
                w2.wait()
                fire_gather(i + 2, b)

        for b in (0, 1):
            i = NCH - 2 + b
            off = base + i * CH
            wait_gather(i, b)
            pltpu.sync_copy(srow[b], sout_hbm.at[pl.ds(off, CH)])
            pltpu.sync_copy(rrow[b], rout_hbm.at[pl.ds(off, CH)])

        @pl.when(wid < NX)
        def _():
            xoff = EMAIN + wid * CH
            pltpu.sync_copy(sx_hbm.at[wid], sxi_v.at[0])
            pltpu.sync_copy(rx_hbm.at[wid], rxi_v.at[0])
            g1 = pltpu.async_copy(node_hbm.at[sxi_v.at[0]], sr0, sg0)
            g2 = pltpu.async_copy(node_hbm.at[rxi_v.at[0]], rr0, sg0)
            g1.wait()
            g2.wait()
            pltpu.sync_copy(sr0, sout_hbm.at[pl.ds(xoff, CH)])
            pltpu.sync_copy(rr0, rout_hbm.at[pl.ds(xoff, CH)])

    @functools.partial(
        pl.kernel,
        out_type=jax.ShapeDtypeStruct((NC, A, H), jnp.float32),
        mesh=mesh,
        scratch_types=[
            pltpu.VMEM((NCH, CH), jnp.int32),
            pltpu.VMEM((1, CH), jnp.int32),
            pltpu.VMEM((CH, H), jnp.float32),
            pltpu.VMEM((CH, H), jnp.float32),
            pltpu.VMEM_SHARED((A, H), jnp.float32),
            pltpu.SemaphoreType.DMA,
            pltpu.SemaphoreType.DMA,
            pltpu.SemaphoreType.DMA,
            pltpu.SemaphoreType.DMA,
        ],
    )
    def _sc_scatter(msg_hbm, rm_hbm, rx_hbm, zero_hbm, out_hbm,
                    ridx_v, rxi_v, mr0, mr1, acc_sh, sl0, sl1, sa0, sa1):
        c = lax.axis_index("c")
        s = lax.axis_index("s")
        wid = s * NC + c
        base = wid * EPW
        r0 = s * RPS
        mrow = (mr0, mr1)
        sls = (sl0, sl1)
        sas = (sa0, sa1)
        pltpu.sync_copy(zero_hbm.at[pl.ds(r0, RPS)], acc_sh.at[pl.ds(r0, RPS)])

        @pl.when(s == 0)
        def _():
            pltpu.sync_copy(zero_hbm.at[pl.ds(TAIL0, TAILN)],
                            acc_sh.at[pl.ds(TAIL0, TAILN)])

        pltpu.sync_copy(rm_hbm.at[wid], ridx_v)
        plsc.subcore_barrier()

        def fire_load(i, b):
            off = base + i * CH
            pltpu.async_copy(msg_hbm.at[pl.ds(off, CH)], mrow[b], sls[b])

        def wait_load(i, b):
            off = base + i * CH
            pltpu.make_async_copy(msg_hbm.at[pl.ds(off, CH)], mrow[b],
                                  sls[b]).wait()

        fire_load(0, 0)
        fire_load(1, 1)

        @pl.loop(0, (NCH - 2) // 2)
        def _(j):
            for b in (0, 1):
                i = 2 * j + b
                wait_load(i, b)
                a = pltpu.async_copy(mrow[b], acc_sh.at[ridx_v.at[i]],
                                     sas[b], add=True)
                a.wait()
                fire_load(i + 2, b)

        for b in (0, 1):
            i = NCH - 2 + b
            wait_load(i, b)
            pltpu.sync_copy(mrow[b], acc_sh.at[ridx_v.at[i]], add=True)

        @pl.when(wid < NX)
        def _():
            xoff = EMAIN + wid * CH
            pltpu.sync_copy(rx_hbm.at[wid], rxi_v.at[0])
            pltpu.sync_copy(msg_hbm.at[pl.ds(xoff, CH)], mr0)
            pltpu.sync_copy(mr0, acc_sh.at[rxi_v.at[0]], add=True)

        plsc.subcore_barrier()
        pltpu.sync_copy(acc_sh.at[pl.ds(r0, RPS)], out_hbm.at[c, pl.ds(r0, RPS)])

        @pl.when(s == 0)
        def _():
            pltpu.sync_copy(acc_sh.at[pl.ds(TAIL0, TAILN)],
                            out_hbm.at[c, pl.ds(TAIL0, TAILN)])

    return _sc_gather, _sc_scatter


BE = 8000


def _mlp_body(sb, rb, p, w1a, w1b, b1, w2, b2, o):
    f32 = jnp.float32
    z = (jnp.dot(sb[...], w1a[...], preferred_element_type=f32)
         + jnp.dot(rb[...], w1b[...], preferred_element_type=f32)
         + b1[...])
    h = jnp.tanh(z)
    m = jnp.tanh(jnp.dot(h, w2[...], preferred_element_type=f32) + b2[...])
    o[...] = m * p[...]


def _tc_mlp(sb, rb, p, w1a, w1b, b1, w2, b2):
    return pl.pallas_call(
        _mlp_body,
        grid=(E // BE,),
        in_specs=[
            pl.BlockSpec((BE, H), lambda i: (i, 0)),
            pl.BlockSpec((BE, H), lambda i: (i, 0)),
            pl.BlockSpec((BE, 1), lambda i: (i, 0)),
            pl.BlockSpec((H, H), lambda i: (0, 0)),
            pl.BlockSpec((H, H), lambda i: (0, 0)),
            pl.BlockSpec((1, H), lambda i: (0, 0)),
            pl.BlockSpec((H, H), lambda i: (0, 0)),
            pl.BlockSpec((1, H), lambda i: (0, 0)),
        ],
        out_specs=pl.BlockSpec((BE, H), lambda i: (i, 0)),
        out_shape=jax.ShapeDtypeStruct((E, H), jnp.float32),
    )(sb, rb, p, w1a, w1b, b1, w2, b2)


def _add_body(a, o):
    o[...] = a[0] + a[1]


def _tc_add(partials):
    return pl.pallas_call(
        _add_body,
        grid=(10,),
        in_specs=[pl.BlockSpec((NC, A // 10, H), lambda i: (0, i, 0))],
        out_specs=pl.BlockSpec((A // 10, H), lambda i: (i, 0)),
        out_shape=jax.ShapeDtypeStruct((A, H), jnp.float32),
    )(partials)


def kernel(node_embedding, edge_probs, send_edges, recv_edges, node_masks,
           W1, b1, W2, b2):
    del node_masks
    x = node_embedding[0]
    p = edge_probs[0, :, 1:2]

    sm = send_edges[:EMAIN].reshape(NW, NCH, CH)
    rm = recv_edges[:EMAIN].reshape(NW, NCH, CH)
    sx = send_edges[EMAIN:].reshape(NX, CH)
    rx = recv_edges[EMAIN:].reshape(NX, CH)

    _sc_gather, _sc_scatter = _sc_kernels()
    sb, rb = _sc_gather(x, sm, rm, sx, rx)

    bh = jnp.bfloat16
    msg = _tc_mlp(sb, rb, p,
                  W1[:H].astype(bh), W1[H:].astype(bh),
                  b1.reshape(1, H), W2.astype(bh), b2.reshape(1, H))
    zeros = jnp.zeros((A, H), jnp.float32)
    partials = _sc_scatter(msg, rm, rx, zeros)
    return _tc_add(partials)[None]
